# Initial kernel scaffold; baseline (speedup 1.0000x reference)
import jax
import jax.numpy as jnp
from jax.experimental import pallas as pl


def kernel(x, lut):
    raise NotImplementedError("write your pallas kernel here")


# speedup vs baseline: 3.5594x; 3.5594x over previous
import functools

import jax
import jax.numpy as jnp
from jax import lax
from jax.experimental import pallas as pl
from jax.experimental.pallas import tpu as pltpu
from jax.experimental.pallas import tpu_sc as plsc

D_MODEL = 64
NUM_CORES = 2
NUM_SUBCORES = 16
NW = NUM_CORES * NUM_SUBCORES
CHUNK = 128
LANES = 16


def _dyn_gather16(v, g):
    return lax.gather(
        v,
        g[:, None],
        dimension_numbers=lax.GatherDimensionNumbers(
            offset_dims=(), collapsed_slice_dims=(0,), start_index_map=(0,)),
        slice_sizes=(1,),
        mode=lax.GatherScatterMode.PROMISE_IN_BOUNDS,
    )


def _sc_embed_body(np_per_w, p_hbm, idx_hbm, out_hbm,
                   raw_v, pidx_v, rows_v, sem):
    wid = lax.axis_index("s") * NUM_CORES + lax.axis_index("c")
    base = wid * np_per_w

    lane = lax.iota(jnp.int32, LANES)
    g_even = (lane * 2) & 15
    g_odd = g_even + 1
    lo_half = lane < 8

    def body(i, carry):
        pair_off = base + i * CHUNK
        pltpu.sync_copy(idx_hbm.at[pl.ds(pair_off * 2, 2 * CHUNK)], raw_v)
        for k in range(CHUNK // LANES):
            v0 = raw_v[pl.ds(2 * LANES * k, LANES)]
            v1 = raw_v[pl.ds(2 * LANES * k + LANES, LANES)]
            e = jnp.where(lo_half, _dyn_gather16(v0, g_even),
                          _dyn_gather16(v1, g_even))
            o = jnp.where(lo_half, _dyn_gather16(v0, g_odd),
                          _dyn_gather16(v1, g_odd))
            pidx_v[pl.ds(LANES * k, LANES)] = e * 16 + o
        pltpu.async_copy(p_hbm.at[pidx_v], rows_v, sem).wait()
        pltpu.sync_copy(rows_v, out_hbm.at[pl.ds(pair_off, CHUNK)])
        return carry

    lax.fori_loop(0, np_per_w // CHUNK, body, 0)


def kernel(x, lut):
    orig_shape = x.shape
    idx = x.reshape(-1).astype(jnp.int32)
    n = idx.shape[0]
    npairs = n // 2
    assert npairs % (NW * CHUNK) == 0
    np_per_w = npairs // NW

    m = lut.shape[0]
    p_tab = jnp.concatenate(
        [jnp.broadcast_to(lut[:, None, :], (m, m, D_MODEL)),
         jnp.broadcast_to(lut[None, :, :], (m, m, D_MODEL))],
        axis=-1).reshape(m * m, 2 * D_MODEL)

    mesh = plsc.VectorSubcoreMesh(core_axis_name="c", subcore_axis_name="s")
    run = pl.kernel(
        functools.partial(_sc_embed_body, np_per_w),
        mesh=mesh,
        out_type=jax.ShapeDtypeStruct((npairs, 2 * D_MODEL), jnp.float32),
        scratch_types=[
            pltpu.VMEM((2 * CHUNK,), jnp.int32),
            pltpu.VMEM((CHUNK,), jnp.int32),
            pltpu.VMEM((CHUNK, 2 * D_MODEL), jnp.float32),
            pltpu.SemaphoreType.DMA,
        ],
    )
    out = run(p_tab, idx)
    return out.reshape(*orig_shape, D_MODEL)


# speedup vs baseline: 3.5827x; 1.0066x over previous
import functools

import jax
import jax.numpy as jnp
from jax import lax
from jax.experimental import pallas as pl
from jax.experimental.pallas import tpu as pltpu
from jax.experimental.pallas import tpu_sc as plsc

D_MODEL = 64
NUM_CORES = 2
NUM_SUBCORES = 16
NW = NUM_CORES * NUM_SUBCORES
CHUNK = 256
LANES = 16


def _dyn_gather16(v, g):
    return lax.gather(
        v,
        g[:, None],
        dimension_numbers=lax.GatherDimensionNumbers(
            offset_dims=(), collapsed_slice_dims=(0,), start_index_map=(0,)),
        slice_sizes=(1,),
        mode=lax.GatherScatterMode.PROMISE_IN_BOUNDS,
    )


def _sc_embed_body(np_per_w, p_hbm, idx_hbm, out_hbm,
                   raw0, raw1, pidx0, pidx1, rows0, rows1,
                   isem0, isem1, gsem0, gsem1, ssem0, ssem1):
    wid = lax.axis_index("s") * NUM_CORES + lax.axis_index("c")
    base = wid * np_per_w
    nch = np_per_w // CHUNK

    raw = (raw0, raw1)
    pidx = (pidx0, pidx1)
    rows = (rows0, rows1)
    isem = (isem0, isem1)
    gsem = (gsem0, gsem1)
    ssem = (ssem0, ssem1)

    lane = lax.iota(jnp.int32, LANES)
    g_even = (lane * 2) & 15
    g_odd = g_even + 1
    lo_half = lane < 8

    def idx_copy(i, b):
        return pltpu.make_async_copy(
            idx_hbm.at[pl.ds((base + i * CHUNK) * 2, 2 * CHUNK)], raw[b],
            isem[b])

    def gather_copy(i, b):
        return pltpu.make_async_copy(p_hbm.at[pidx[b]], rows[b], gsem[b])

    def scatter_copy(i, b):
        return pltpu.make_async_copy(
            rows[b], out_hbm.at[pl.ds(base + i * CHUNK, CHUNK)], ssem[b])

    idx_copy(0, 0).start()
    idx_copy(1, 1).start()

    def body(j, carry):
        for b in (0, 1):
            i = 2 * j + b
            idx_copy(i, b).wait()
            for k in range(CHUNK // LANES):
                v0 = raw[b][pl.ds(2 * LANES * k, LANES)]
                v1 = raw[b][pl.ds(2 * LANES * k + LANES, LANES)]
                e = jnp.where(lo_half, _dyn_gather16(v0, g_even),
                              _dyn_gather16(v1, g_even))
                o = jnp.where(lo_half, _dyn_gather16(v0, g_odd),
                              _dyn_gather16(v1, g_odd))
                pidx[b][pl.ds(LANES * k, LANES)] = e * 16 + o
            @pl.when(j < nch // 2 - 1)
            def _():
                idx_copy(i + 2, b).start()
            @pl.when(j >= 1)
            def _():
                scatter_copy(i - 2, b).wait()
            gather_copy(i, b).start()
            gather_copy(i, b).wait()
            scatter_copy(i, b).start()
        return carry

    lax.fori_loop(0, nch // 2, body, 0)
    scatter_copy(nch - 2, 0).wait()
    scatter_copy(nch - 1, 1).wait()


def kernel(x, lut):
    orig_shape = x.shape
    idx = x.reshape(-1).astype(jnp.int32)
    n = idx.shape[0]
    npairs = n // 2
    assert npairs % (NW * CHUNK * 2) == 0
    np_per_w = npairs // NW

    m = lut.shape[0]
    p_tab = jnp.concatenate(
        [jnp.broadcast_to(lut[:, None, :], (m, m, D_MODEL)),
         jnp.broadcast_to(lut[None, :, :], (m, m, D_MODEL))],
        axis=-1).reshape(m * m, 2 * D_MODEL)

    mesh = plsc.VectorSubcoreMesh(core_axis_name="c", subcore_axis_name="s")
    run = pl.kernel(
        functools.partial(_sc_embed_body, np_per_w),
        mesh=mesh,
        out_type=jax.ShapeDtypeStruct((npairs, 2 * D_MODEL), jnp.float32),
        scratch_types=[
            pltpu.VMEM((2 * CHUNK,), jnp.int32),
            pltpu.VMEM((2 * CHUNK,), jnp.int32),
            pltpu.VMEM((CHUNK,), jnp.int32),
            pltpu.VMEM((CHUNK,), jnp.int32),
            pltpu.VMEM((CHUNK, 2 * D_MODEL), jnp.float32),
            pltpu.VMEM((CHUNK, 2 * D_MODEL), jnp.float32),
            pltpu.SemaphoreType.DMA,
            pltpu.SemaphoreType.DMA,
            pltpu.SemaphoreType.DMA,
            pltpu.SemaphoreType.DMA,
            pltpu.SemaphoreType.DMA,
            pltpu.SemaphoreType.DMA,
        ],
    )
    out = run(p_tab, idx)
    return out.reshape(*orig_shape, D_MODEL)
